# baseline (speedup 1.0000x reference)
import functools

import jax
import jax.numpy as jnp
from jax import lax
from jax.experimental import pallas as pl
from jax.experimental.pallas import tpu as pltpu
from jax.experimental.pallas import tpu_sc as plsc

_LANES = 16
_BLK = 128
_DEPTH = 4


def _build_lookup(batch, hist, dim):
    nc, ns = 2, 16
    nw = nc * ns
    n_blk = batch // _BLK
    blk_per_w = n_blk // nw
    assert batch == nw * blk_per_w * _BLK and hist % 2 == 0 and dim % 8 == 0
    kb = dim // 8
    grp = _BLK // _LANES

    mesh = plsc.VectorSubcoreMesh(
        core_axis_name="c", subcore_axis_name="s", num_cores=nc, num_subcores=ns
    )

    @functools.partial(
        pl.kernel,
        out_type=jax.ShapeDtypeStruct((hist, kb, n_blk, 8, _BLK), jnp.float32),
        mesh=mesh,
        compiler_params=pltpu.CompilerParams(
            use_tc_tiling_on_sc=False, needs_layout_passes=False
        ),
        scratch_types=[
            pltpu.VMEM((_BLK, hist), jnp.int32),
            [pltpu.VMEM((_BLK,), jnp.int32) for _ in range(_DEPTH)],
            [pltpu.VMEM((_BLK, dim), jnp.float32) for _ in range(_DEPTH)],
            [pltpu.VMEM((dim, _BLK + 1), jnp.float32) for _ in range(_DEPTH)],
            [pltpu.SemaphoreType.DMA for _ in range(_DEPTH)],
            [pltpu.SemaphoreType.DMA for _ in range(_DEPTH)],
            pltpu.SMEM((_DEPTH,), jnp.int32),
        ],
    )
    def body(
        ids_hbm, table_hbm, out_hbm,
        ids_v, idxs, rowss, tbs, gss, oss, zcnt,
    ):
        wid = lax.axis_index("s") * nc + lax.axis_index("c")
        lane_iota = lax.iota(jnp.int32, _LANES)
        ones_i = jnp.full((_LANES,), 1, jnp.int32)
        fours_i = jnp.full((_LANES,), 4, jnp.int32)
        zeros_i = jnp.zeros((_LANES,), jnp.int32)
        zeros_f = jnp.zeros((_LANES,), jnp.float32)

        def build_idx(h, idx_ref, slot, zcnt):
            col = jnp.full((_LANES,), h, jnp.int32)
            accv = zeros_i
            for g in range(grp):
                row = jnp.full((_LANES,), g * _LANES, jnp.int32) + lane_iota
                v = plsc.load_gather(ids_v, [row, col])
                adj = jnp.maximum(v - ones_i, zeros_i)
                idx_ref[pl.ds(g * _LANES, _LANES)] = adj * fours_i
                accv = accv + jnp.where(v == zeros_i, ones_i, zeros_i)
            zcnt[slot] = jnp.sum(accv)

        def fire_gather(idx_ref, rows_ref, sem):
            pltpu.async_copy(table_hbm.at[idx_ref], rows_ref, sem)

        def wait_gather(idx_ref, rows_ref, sem):
            pltpu.make_async_copy(table_hbm.at[idx_ref], rows_ref, sem).wait()

        def fixup(h, rows_ref, slot, zcnt):
            @pl.when(zcnt[slot] > 0)
            def _fix_col():
                col = jnp.full((_LANES,), h, jnp.int32)
                for g in range(grp):
                    row = jnp.full((_LANES,), g * _LANES, jnp.int32) + lane_iota
                    v = plsc.load_gather(ids_v, [row, col])
                    mask = v == zeros_i
                    for k in range(dim):
                        plsc.store_scatter(
                            rows_ref,
                            [row, jnp.full((_LANES,), k, jnp.int32)],
                            zeros_f,
                            mask=mask,
                        )

        def transpose(rows_ref, tb_ref):
            kidx = [
                jnp.full((_LANES,), c * _LANES, jnp.int32) + lane_iota
                for c in range(dim // _LANES)
            ]
            for r in range(_BLK):
                colr = jnp.full((_LANES,), r, jnp.int32)
                for c in range(dim // _LANES):
                    v = rows_ref[r, pl.ds(c * _LANES, _LANES)]
                    plsc.store_scatter(tb_ref, [kidx[c], colr], v)

        def fire_out(h, blk, tb_ref, sem):
            for t in range(kb):
                pltpu.async_copy(
                    tb_ref.at[pl.ds(t * 8, 8), pl.ds(0, _BLK)],
                    out_hbm.at[h, t, blk],
                    sem,
                )

        def drain_out(h, blk, tb_ref, sem):
            for t in range(kb):
                pltpu.make_async_copy(
                    tb_ref.at[pl.ds(t * 8, 8), pl.ds(0, _BLK)],
                    out_hbm.at[h, t, blk],
                    sem,
                ).wait()

        n_full = hist // _DEPTH
        tail = hist - n_full * _DEPTH

        def unit(h, g, j, blk):
            wait_gather(idxs[j], rowss[j], gss[j])
            fixup(h, rowss[j], j, zcnt)

            @pl.when(g >= 1)
            def _d():
                drain_out(h, blk, tbs[j], oss[j])

            transpose(rowss[j], tbs[j])
            fire_out(h, blk, tbs[j], oss[j])

            @pl.when(h + _DEPTH < hist)
            def _next():
                build_idx(h + _DEPTH, idxs[j], j, zcnt)
                fire_gather(idxs[j], rowss[j], gss[j])

        def blk_body(bi, carry):
            blk = wid * blk_per_w + bi
            pltpu.sync_copy(ids_hbm.at[pl.ds(blk * _BLK, _BLK)], ids_v)

            for j in range(_DEPTH):
                build_idx(j, idxs[j], j, zcnt)
                fire_gather(idxs[j], rowss[j], gss[j])

            def grp_body(g, carry2):
                for j in range(_DEPTH):
                    unit(_DEPTH * g + j, g, j, blk)
                return carry2

            lax.fori_loop(0, n_full, grp_body, 0)
            for j in range(tail):
                h = n_full * _DEPTH + j
                wait_gather(idxs[j], rowss[j], gss[j])
                fixup(h, rowss[j], j, zcnt)
                drain_out(h, blk, tbs[j], oss[j])
                transpose(rowss[j], tbs[j])
                fire_out(h, blk, tbs[j], oss[j])
            for j in range(_DEPTH):
                h = hist - _DEPTH + j
                drain_out(h, blk, tbs[j], oss[j])
            return carry

        lax.fori_loop(0, blk_per_w, blk_body, 0)

    return body


def kernel(ids, embs):
    batch, hist = ids.shape
    vocab, dim = embs.shape
    table = jnp.pad(embs, ((0, 0), (0, 128 - dim))).reshape(vocab * (128 // dim), dim)
    lookup = _build_lookup(batch, hist, dim)
    x = lookup(ids.astype(jnp.int32), table)
    return x.transpose(2, 4, 0, 1, 3).reshape(batch, hist, dim)


# speedup vs baseline: 1.1506x; 1.1506x over previous
import functools

import jax
import jax.numpy as jnp
from jax import lax
from jax.experimental import pallas as pl
from jax.experimental.pallas import tpu as pltpu
from jax.experimental.pallas import tpu_sc as plsc

_LANES = 16
_BLK = 128


def _build_lookup(batch, hist, dim):
    nc, ns = 2, 16
    nw = nc * ns
    n_blk = batch // _BLK
    blk_per_w = n_blk // nw
    assert batch == nw * blk_per_w * _BLK and hist % 2 == 0 and dim % 8 == 0
    kb = dim // 8
    grp = _BLK // _LANES

    mesh = plsc.VectorSubcoreMesh(
        core_axis_name="c", subcore_axis_name="s", num_cores=nc, num_subcores=ns
    )

    @functools.partial(
        pl.kernel,
        out_type=jax.ShapeDtypeStruct((hist, kb, n_blk, 8, _BLK), jnp.float32),
        mesh=mesh,
        compiler_params=pltpu.CompilerParams(
            use_tc_tiling_on_sc=False, needs_layout_passes=False
        ),
        scratch_types=[
            pltpu.VMEM((_BLK, hist), jnp.int32),
            pltpu.VMEM((_BLK,), jnp.int32),
            pltpu.VMEM((_BLK,), jnp.int32),
            pltpu.VMEM((_BLK, dim), jnp.float32),
            pltpu.VMEM((_BLK, dim), jnp.float32),
            pltpu.VMEM((dim, _BLK + 1), jnp.float32),
            pltpu.VMEM((dim, _BLK + 1), jnp.float32),
            pltpu.SemaphoreType.DMA,
            pltpu.SemaphoreType.DMA,
            pltpu.SemaphoreType.DMA,
            pltpu.SemaphoreType.DMA,
            pltpu.SMEM((2,), jnp.int32),
        ],
    )
    def body(
        ids_hbm, table_hbm, out_hbm,
        ids_v, idx0, idx1, rows0, rows1, tb0, tb1, gs0, gs1, os0, os1, zcnt,
    ):
        wid = lax.axis_index("s") * nc + lax.axis_index("c")
        lane_iota = lax.iota(jnp.int32, _LANES)
        ones_i = jnp.full((_LANES,), 1, jnp.int32)
        fours_i = jnp.full((_LANES,), 4, jnp.int32)
        zeros_i = jnp.zeros((_LANES,), jnp.int32)
        zeros_f = jnp.zeros((_LANES,), jnp.float32)

        def build_idx(h, idx_ref, slot, zcnt):
            col = jnp.full((_LANES,), h, jnp.int32)
            accv = zeros_i
            for g in range(grp):
                row = jnp.full((_LANES,), g * _LANES, jnp.int32) + lane_iota
                v = plsc.load_gather(ids_v, [row, col])
                adj = jnp.maximum(v - ones_i, zeros_i)
                idx_ref[pl.ds(g * _LANES, _LANES)] = adj * fours_i
                accv = accv + jnp.where(v == zeros_i, ones_i, zeros_i)
            zcnt[slot] = jnp.sum(accv)

        def fire_gather(idx_ref, rows_ref, sem):
            pltpu.async_copy(table_hbm.at[idx_ref], rows_ref, sem)

        def wait_gather(idx_ref, rows_ref, sem):
            pltpu.make_async_copy(table_hbm.at[idx_ref], rows_ref, sem).wait()

        def fixup(h, rows_ref, slot, zcnt):
            @pl.when(zcnt[slot] > 0)
            def _fix_col():
                col = jnp.full((_LANES,), h, jnp.int32)
                for g in range(grp):
                    row = jnp.full((_LANES,), g * _LANES, jnp.int32) + lane_iota
                    v = plsc.load_gather(ids_v, [row, col])
                    mask = v == zeros_i
                    for k in range(dim):
                        plsc.store_scatter(
                            rows_ref,
                            [row, jnp.full((_LANES,), k, jnp.int32)],
                            zeros_f,
                            mask=mask,
                        )

        def transpose(rows_ref, tb_ref):
            kidx = [
                jnp.full((_LANES,), c * _LANES, jnp.int32) + lane_iota
                for c in range(dim // _LANES)
            ]
            for r in range(_BLK):
                colr = jnp.full((_LANES,), r, jnp.int32)
                for c in range(dim // _LANES):
                    v = rows_ref[r, pl.ds(c * _LANES, _LANES)]
                    plsc.store_scatter(tb_ref, [kidx[c], colr], v)

        def fire_out(h, blk, tb_ref, sem):
            for t in range(kb):
                pltpu.async_copy(
                    tb_ref.at[pl.ds(t * 8, 8), pl.ds(0, _BLK)],
                    out_hbm.at[h, t, blk],
                    sem,
                )

        def drain_out(h, blk, tb_ref, sem):
            for t in range(kb):
                pltpu.make_async_copy(
                    tb_ref.at[pl.ds(t * 8, 8), pl.ds(0, _BLK)],
                    out_hbm.at[h, t, blk],
                    sem,
                ).wait()

        def blk_body(bi, carry):
            blk = wid * blk_per_w + bi
            pltpu.sync_copy(ids_hbm.at[pl.ds(blk * _BLK, _BLK)], ids_v)

            build_idx(0, idx0, 0, zcnt)
            fire_gather(idx0, rows0, gs0)

            def pair_body(g, carry2):
                h0 = 2 * g
                h1 = 2 * g + 1

                build_idx(h1, idx1, 1, zcnt)
                fire_gather(idx1, rows1, gs1)

                wait_gather(idx0, rows0, gs0)
                fixup(h0, rows0, 0, zcnt)

                @pl.when(g >= 1)
                def _d0():
                    drain_out(h0, blk, tb0, os0)

                transpose(rows0, tb0)
                fire_out(h0, blk, tb0, os0)
                wait_gather(idx1, rows1, gs1)
                fixup(h1, rows1, 1, zcnt)

                @pl.when(g + 1 < hist // 2)
                def _next():
                    build_idx(h1 + 1, idx0, 0, zcnt)
                    fire_gather(idx0, rows0, gs0)

                @pl.when(g >= 1)
                def _d1():
                    drain_out(h1, blk, tb1, os1)

                transpose(rows1, tb1)
                fire_out(h1, blk, tb1, os1)
                return carry2

            lax.fori_loop(0, hist // 2, pair_body, 0)
            drain_out(hist - 2, blk, tb0, os0)
            drain_out(hist - 1, blk, tb1, os1)
            return carry

        lax.fori_loop(0, blk_per_w, blk_body, 0)

    return body


def kernel(ids, embs):
    batch, hist = ids.shape
    vocab, dim = embs.shape
    table = jnp.pad(embs, ((0, 0), (0, 128 - dim))).reshape(vocab * (128 // dim), dim)
    lookup = _build_lookup(batch, hist, dim)
    x = lookup(ids.astype(jnp.int32), table)
    return x.transpose(2, 4, 0, 1, 3).reshape(batch, hist, dim)
